# baseline (speedup 1.0000x reference)
import functools

import numpy as np
import jax
import jax.numpy as jnp
from jax import lax
from jax.experimental import pallas as pl
from jax.experimental.pallas import tpu as pltpu
from jax.experimental.pallas import tpu_sc as plsc

_B, _S, _D = 4096, 200, 64
_N = _B * _S
_V = 1000000
_NC, _NS, _L = 2, 16, 16
_NW = _NC * _NS
_CHUNK = 128
_NT = (_V + _CHUNK - 1) // _CHUNK
_VP = _NT * _CHUNK // 2
_NBUF = 2
_BPW = _B // _NW


def _pos_enc8() -> np.ndarray:
    d = np.arange(_D)[np.newaxis, :]
    d = 1.0 / np.power(10000, 2 * (d // 2) / np.float32(_D))
    t = np.arange(_S)[:, np.newaxis] * d
    t = np.concatenate([np.sin(t[:, 0::2]), np.cos(t[:, 1::2])], axis=-1)
    return (t * 8.0).astype(np.float32).reshape(-1)


def _make_format_kernel():
    mesh = plsc.VectorSubcoreMesh(
        core_axis_name="c", subcore_axis_name="s",
        num_cores=_NC, num_subcores=_NS,
    )

    @functools.partial(
        pl.kernel,
        out_type=jax.ShapeDtypeStruct((_VP, _CHUNK), jnp.float32),
        mesh=mesh,
        scratch_types=[
            pltpu.VMEM((_NBUF, _D, _CHUNK), jnp.float32),
            pltpu.VMEM((_NBUF, _D, _CHUNK), jnp.float32),
            pltpu.SemaphoreType.DMA,
            pltpu.SemaphoreType.DMA,
            pltpu.SemaphoreType.DMA,
            pltpu.SemaphoreType.DMA,
        ],
        compiler_params=pltpu.CompilerParams(
            use_tc_tiling_on_sc=True, disable_bounds_checks=True,
            needs_layout_passes=False),
    )
    def body(embt_hbm, tab_hbm, e_v, t_v, semi0, semi1, semo0, semo1):
        semi = (semi0, semi1)
        semo = (semo0, semo1)
        wid = lax.axis_index("s") * _NC + lax.axis_index("c")
        nt = (_NT + _NW - 1 - wid) // _NW

        def start(i, b):
            t = wid + i * _NW
            pltpu.async_copy(
                embt_hbm.at[:, pl.ds(t * _CHUNK, _CHUNK)], e_v.at[b],
                semi[b])

        def wait_in(i, b):
            t = wid + i * _NW
            pltpu.make_async_copy(
                embt_hbm.at[:, pl.ds(t * _CHUNK, _CHUNK)], e_v.at[b],
                semi[b]).wait()

        def out_desc(i, b):
            t = wid + i * _NW
            return pltpu.make_async_copy(
                t_v.at[b], tab_hbm.at[pl.ds(t * _D, _D)], semo[b])

        for b in range(_NBUF):
            start(b, b)

        iota = lax.iota(jnp.int32, _L)
        rows = [iota + g * _L for g in range(_D // _L)]

        @pl.loop(0, 246, step=_NBUF)
        def _tiles(c):
            for b in range(_NBUF):
                i = c + b

                @pl.when(i < nt)
                def _():
                    wait_in(i, b)

                    @pl.when(i >= _NBUF)
                    def _():
                        out_desc(i - _NBUF, b).wait()

                    @pl.loop(0, _D, unroll=4)
                    def _row(r):
                        j0 = jnp.full((_L,), 2 * r, jnp.int32)
                        j1 = jnp.full((_L,), 2 * r + 1, jnp.int32)
                        for g in range(_D // _L):
                            t_v[b, r, pl.ds(g * _L, _L)] = plsc.load_gather(
                                e_v.at[b], [rows[g], j0])
                            t_v[b, r, pl.ds(_D + g * _L, _L)] = (
                                plsc.load_gather(e_v.at[b], [rows[g], j1]))

                    out_desc(i, b).start()

                @pl.when(i + _NBUF < nt)
                def _():
                    start(i + _NBUF, b)

        for b in range(_NBUF):
            out_desc(0, b).wait()

    return body


def _make_embed_kernel():
    mesh = plsc.VectorSubcoreMesh(
        core_axis_name="c", subcore_axis_name="s",
        num_cores=_NC, num_subcores=_NS,
    )

    @functools.partial(
        pl.kernel,
        out_type=jax.ShapeDtypeStruct((_S, _D, _B), jnp.float32),
        mesh=mesh,
        scratch_types=[
            pltpu.VMEM((_S, _CHUNK), jnp.int32),
            pltpu.VMEM((_NBUF, _CHUNK), jnp.int32),
            pltpu.VMEM((_NBUF, _CHUNK), jnp.int32),
            pltpu.VMEM((_S * _D,), jnp.float32),
            pltpu.VMEM((_NBUF, _CHUNK, _CHUNK), jnp.float32),
            pltpu.VMEM((_NBUF, _D, _CHUNK), jnp.float32),
            pltpu.SemaphoreType.DMA,
            pltpu.SemaphoreType.DMA,
            pltpu.SemaphoreType.DMA,
            pltpu.SemaphoreType.DMA,
        ],
        compiler_params=pltpu.CompilerParams(
            use_tc_tiling_on_sc=True, needs_layout_passes=False),
    )
    def body(yt_hbm, pos_hbm, tab_hbm, out_hbm, idx_v, ix2_v, par_v, pos_v,
             buf_v, o_v, semi0, semi1, semo0, semo1):
        semi = (semi0, semi1)
        semo = (semo0, semo1)
        wid = lax.axis_index("s") * _NC + lax.axis_index("c")
        b0 = wid * _BPW
        pltpu.sync_copy(yt_hbm.at[:, pl.ds(b0, _BPW)], idx_v)
        pltpu.sync_copy(pos_hbm, pos_v)

        def prep_idx(s, b):
            for g in range(_CHUNK // _L):
                sl = pl.ds(g * _L, _L)
                iv = idx_v[s, sl]
                ix2_v[b, sl] = iv >> 1
                par_v[b, sl] = (iv & 1) << 6

        def start(s, b):
            pltpu.async_copy(
                tab_hbm.at[ix2_v.at[b]], buf_v.at[b], semi[b])

        def wait_in(b):
            pltpu.make_async_copy(
                tab_hbm.at[ix2_v.at[b]], buf_v.at[b], semi[b]).wait()

        def out_desc(s, b):
            return pltpu.make_async_copy(
                o_v.at[b], out_hbm.at[s, :, pl.ds(b0, _BPW)], semo[b])

        for b in range(_NBUF):
            prep_idx(b, b)
            start(b, b)

        iota = lax.iota(jnp.int32, _L)
        rows = [iota + g * _L for g in range(_CHUNK // _L)]

        @pl.loop(0, _S, step=_NBUF)
        def _chunks(c):
            for b in range(_NBUF):
                s = c + b
                wait_in(b)

                @pl.when(s >= _NBUF)
                def _():
                    out_desc(s - _NBUF, b).wait()

                pars = [par_v[b, pl.ds(g * _L, _L)]
                        for g in range(_CHUNK // _L)]

                @pl.loop(0, _D, unroll=2)
                def _col(d):
                    pv = plsc.load_gather(
                        pos_v, [jnp.full((_L,), s * _D + d, jnp.int32)])
                    for g in range(_CHUNK // _L):
                        vals = plsc.load_gather(
                            buf_v.at[b], [rows[g], pars[g] + d])
                        o_v[b, d, pl.ds(g * _L, _L)] = vals * 8.0 + pv

                out_desc(s, b).start()

                nxt = s + _NBUF

                @pl.when(nxt < _S)
                def _():
                    prep_idx(nxt, b)
                    start(nxt, b)

        for b in range(_NBUF):
            out_desc(0, b).wait()

    return body


_FORMAT_KERNEL = _make_format_kernel()
_EMBED_KERNEL = _make_embed_kernel()
_POS8 = _pos_enc8()


def kernel(y, lens, emb):
    table = _FORMAT_KERNEL(emb.T)
    out3 = _EMBED_KERNEL(y.T, jnp.asarray(_POS8), table)
    return jnp.transpose(out3, (2, 0, 1)), lens


# speedup vs baseline: 2.0323x; 2.0323x over previous
import functools

import numpy as np
import jax
import jax.numpy as jnp
from jax import lax
from jax.experimental import pallas as pl
from jax.experimental.pallas import tpu as pltpu
from jax.experimental.pallas import tpu_sc as plsc

_B, _S, _D = 4096, 200, 64
_N = _B * _S
_V = 1000000
_NC, _NS, _L = 2, 16, 16
_NW = _NC * _NS
_CHUNK = 128
_NT = (_V + _CHUNK - 1) // _CHUNK
_VP = _NT * _CHUNK // 2
_NBUF = 2
_BPW = _B // _NW


def _pos_enc8() -> np.ndarray:
    d = np.arange(_D)[np.newaxis, :]
    d = 1.0 / np.power(10000, 2 * (d // 2) / np.float32(_D))
    t = np.arange(_S)[:, np.newaxis] * d
    t = np.concatenate([np.sin(t[:, 0::2]), np.cos(t[:, 1::2])], axis=-1)
    return (t * 8.0).astype(np.float32).reshape(-1)


def _make_format_kernel():
    mesh = plsc.VectorSubcoreMesh(
        core_axis_name="c", subcore_axis_name="s",
        num_cores=_NC, num_subcores=_NS,
    )

    @functools.partial(
        pl.kernel,
        out_type=jax.ShapeDtypeStruct((_VP, _CHUNK), jnp.float32),
        mesh=mesh,
        scratch_types=[
            pltpu.VMEM((_NBUF, _D, _CHUNK), jnp.float32),
            pltpu.VMEM((_NBUF, _D, _CHUNK), jnp.float32),
            pltpu.SemaphoreType.DMA,
            pltpu.SemaphoreType.DMA,
            pltpu.SemaphoreType.DMA,
            pltpu.SemaphoreType.DMA,
        ],
        compiler_params=pltpu.CompilerParams(
            use_tc_tiling_on_sc=True, disable_bounds_checks=True,
            needs_layout_passes=False),
    )
    def body(embt_hbm, tab_hbm, e_v, t_v, semi0, semi1, semo0, semo1):
        semi = (semi0, semi1)
        semo = (semo0, semo1)
        wid = lax.axis_index("s") * _NC + lax.axis_index("c")
        nt = (_NT + _NW - 1 - wid) // _NW

        def start(i, b):
            t = wid + i * _NW
            pltpu.async_copy(
                embt_hbm.at[:, pl.ds(t * _CHUNK, _CHUNK)], e_v.at[b],
                semi[b])

        def wait_in(i, b):
            t = wid + i * _NW
            pltpu.make_async_copy(
                embt_hbm.at[:, pl.ds(t * _CHUNK, _CHUNK)], e_v.at[b],
                semi[b]).wait()

        def out_desc(i, b):
            t = wid + i * _NW
            return pltpu.make_async_copy(
                t_v.at[b], tab_hbm.at[pl.ds(t * _D, _D)], semo[b])

        for b in range(_NBUF):
            start(b, b)

        iota = lax.iota(jnp.int32, _L)
        rot = [(iota + k) & (_L - 1) for k in range(_L)]
        cvec = 2 * (iota & 7) + (iota >> 3)
        r8 = iota & 7
        h64 = (iota >> 3) * _D

        @pl.loop(0, 246, step=_NBUF)
        def _tiles(c):
            for b in range(_NBUF):
                i = c + b

                @pl.when(i < nt)
                def _():
                    wait_in(i, b)

                    @pl.when(i >= _NBUF)
                    def _():
                        out_desc(i - _NBUF, b).wait()

                    @pl.loop(0, 8)
                    def _rb(rb):
                        srccol = jnp.full((_L,), 2 * 8, jnp.int32) * rb + cvec
                        dstrow = jnp.full((_L,), 8, jnp.int32) * rb + r8
                        for db in range(_D // _L):
                            for k in range(_L):
                                srcrow = rot[k] + (db * _L)
                                v = plsc.load_gather(
                                    e_v.at[b], [srcrow, srccol])
                                plsc.store_scatter(
                                    t_v.at[b], [dstrow, h64 + srcrow], v)

                    out_desc(i, b).start()

                @pl.when(i + _NBUF < nt)
                def _():
                    start(i + _NBUF, b)

        for b in range(_NBUF):
            out_desc(0, b).wait()

    return body


def _make_embed_kernel():
    mesh = plsc.VectorSubcoreMesh(
        core_axis_name="c", subcore_axis_name="s",
        num_cores=_NC, num_subcores=_NS,
    )

    @functools.partial(
        pl.kernel,
        out_type=jax.ShapeDtypeStruct((_S, _D, _B), jnp.float32),
        mesh=mesh,
        scratch_types=[
            pltpu.VMEM((_S, _CHUNK), jnp.int32),
            pltpu.VMEM((_NBUF, _CHUNK), jnp.int32),
            pltpu.VMEM((_NBUF, _CHUNK), jnp.int32),
            pltpu.VMEM((_S * _D,), jnp.float32),
            pltpu.VMEM((_NBUF, _CHUNK, _CHUNK), jnp.float32),
            pltpu.VMEM((_NBUF, _D, _CHUNK), jnp.float32),
            pltpu.SemaphoreType.DMA,
            pltpu.SemaphoreType.DMA,
            pltpu.SemaphoreType.DMA,
            pltpu.SemaphoreType.DMA,
        ],
        compiler_params=pltpu.CompilerParams(
            use_tc_tiling_on_sc=True, needs_layout_passes=False),
    )
    def body(yt_hbm, pos_hbm, tab_hbm, out_hbm, idx_v, ix2_v, par_v, pos_v,
             buf_v, o_v, semi0, semi1, semo0, semo1):
        semi = (semi0, semi1)
        semo = (semo0, semo1)
        wid = lax.axis_index("s") * _NC + lax.axis_index("c")
        b0 = wid * _BPW
        pltpu.sync_copy(yt_hbm.at[:, pl.ds(b0, _BPW)], idx_v)
        pltpu.sync_copy(pos_hbm, pos_v)

        def prep_idx(s, b):
            for g in range(_CHUNK // _L):
                sl = pl.ds(g * _L, _L)
                iv = idx_v[s, sl]
                ix2_v[b, sl] = iv >> 1
                par_v[b, sl] = (iv & 1) << 6

        def start(s, b):
            pltpu.async_copy(
                tab_hbm.at[ix2_v.at[b]], buf_v.at[b], semi[b])

        def wait_in(b):
            pltpu.make_async_copy(
                tab_hbm.at[ix2_v.at[b]], buf_v.at[b], semi[b]).wait()

        def out_desc(s, b):
            return pltpu.make_async_copy(
                o_v.at[b], out_hbm.at[s, :, pl.ds(b0, _BPW)], semo[b])

        for b in range(_NBUF):
            prep_idx(b, b)
            start(b, b)

        iota = lax.iota(jnp.int32, _L)
        rot = [(iota + k) & (_L - 1) for k in range(_L)]

        @pl.loop(0, _S, step=_NBUF)
        def _chunks(c):
            for b in range(_NBUF):
                s = c + b
                wait_in(b)

                @pl.when(s >= _NBUF)
                def _():
                    out_desc(s - _NBUF, b).wait()

                for db in range(_D // _L):
                    pbase = jnp.full((_L,), s * _D + db * _L, jnp.int32)

                    @pl.loop(0, _CHUNK // _L)
                    def _jb(jb):
                        jvec = jnp.full((_L,), _L, jnp.int32) * jb + iota
                        parv = par_v[b, pl.ds(jb * _L, _L)]
                        for k in range(_L):
                            drow = rot[k] + (db * _L)
                            v = plsc.load_gather(
                                buf_v.at[b], [jvec, parv + drow])
                            pv = plsc.load_gather(pos_v, [pbase + rot[k]])
                            plsc.store_scatter(
                                o_v.at[b], [drow, jvec], v * 8.0 + pv)

                out_desc(s, b).start()

                nxt = s + _NBUF

                @pl.when(nxt < _S)
                def _():
                    prep_idx(nxt, b)
                    start(nxt, b)

        for b in range(_NBUF):
            out_desc(0, b).wait()

    return body


_FORMAT_KERNEL = _make_format_kernel()
_EMBED_KERNEL = _make_embed_kernel()
_POS8 = _pos_enc8()


def kernel(y, lens, emb):
    table = _FORMAT_KERNEL(emb.T)
    out3 = _EMBED_KERNEL(y.T, jnp.asarray(_POS8), table)
    return jnp.transpose(out3, (2, 0, 1)), lens


# speedup vs baseline: 2.0802x; 1.0235x over previous
import functools

import numpy as np
import jax
import jax.numpy as jnp
from jax import lax
from jax.experimental import pallas as pl
from jax.experimental.pallas import tpu as pltpu
from jax.experimental.pallas import tpu_sc as plsc

_B, _S, _D = 4096, 200, 64
_N = _B * _S
_V = 1000000
_NC, _NS, _L = 2, 16, 16
_NW = _NC * _NS
_CHUNK = 128
_NT = (_V + _CHUNK - 1) // _CHUNK
_VP = _NT * _CHUNK // 2
_NBUF = 2
_BPW = _B // _NW


def _pos_enc8() -> np.ndarray:
    d = np.arange(_D)[np.newaxis, :]
    d = 1.0 / np.power(10000, 2 * (d // 2) / np.float32(_D))
    t = np.arange(_S)[:, np.newaxis] * d
    t = np.concatenate([np.sin(t[:, 0::2]), np.cos(t[:, 1::2])], axis=-1)
    return (t * 8.0).astype(np.float32).reshape(-1)


def _make_format_kernel():
    mesh = plsc.VectorSubcoreMesh(
        core_axis_name="c", subcore_axis_name="s",
        num_cores=_NC, num_subcores=_NS,
    )

    @functools.partial(
        pl.kernel,
        out_type=jax.ShapeDtypeStruct((_VP, _CHUNK), jnp.float32),
        mesh=mesh,
        scratch_types=[
            pltpu.VMEM((_NBUF, _D, _CHUNK), jnp.float32),
            pltpu.VMEM((_NBUF, _D, _CHUNK), jnp.float32),
            pltpu.SemaphoreType.DMA,
            pltpu.SemaphoreType.DMA,
            pltpu.SemaphoreType.DMA,
            pltpu.SemaphoreType.DMA,
        ],
        compiler_params=pltpu.CompilerParams(
            use_tc_tiling_on_sc=True, disable_bounds_checks=True,
            needs_layout_passes=False),
    )
    def body(embt_hbm, tab_hbm, e_v, t_v, semi0, semi1, semo0, semo1):
        semi = (semi0, semi1)
        semo = (semo0, semo1)
        wid = lax.axis_index("s") * _NC + lax.axis_index("c")
        nt = (_NT + _NW - 1 - wid) // _NW

        def start(i, b):
            t = wid + i * _NW
            pltpu.async_copy(
                embt_hbm.at[:, pl.ds(t * _CHUNK, _CHUNK)], e_v.at[b],
                semi[b])

        def wait_in(i, b):
            t = wid + i * _NW
            pltpu.make_async_copy(
                embt_hbm.at[:, pl.ds(t * _CHUNK, _CHUNK)], e_v.at[b],
                semi[b]).wait()

        def out_desc(i, b):
            t = wid + i * _NW
            return pltpu.make_async_copy(
                t_v.at[b], tab_hbm.at[pl.ds(t * _D, _D)], semo[b])

        for b in range(_NBUF):
            start(b, b)

        iota = lax.iota(jnp.int32, _L)
        rot = [(iota + k) & (_L - 1) for k in range(_L)]
        cvec = 2 * (iota & 7) + (iota >> 3)
        r8 = iota & 7
        h64 = (iota >> 3) * _D

        @pl.loop(0, 246, step=_NBUF)
        def _tiles(c):
            for b in range(_NBUF):
                i = c + b

                @pl.when(i < nt)
                def _():
                    wait_in(i, b)

                    @pl.when(i >= _NBUF)
                    def _():
                        out_desc(i - _NBUF, b).wait()

                    @pl.loop(0, 8)
                    def _rb(rb):
                        srccol = jnp.full((_L,), 2 * 8, jnp.int32) * rb + cvec
                        dstrow = jnp.full((_L,), 8, jnp.int32) * rb + r8
                        for db in range(_D // _L):
                            for k in range(_L):
                                srcrow = rot[k] + (db * _L)
                                v = plsc.load_gather(
                                    e_v.at[b], [srcrow, srccol])
                                plsc.store_scatter(
                                    t_v.at[b], [dstrow, h64 + srcrow], v)

                    out_desc(i, b).start()

                @pl.when(i + _NBUF < nt)
                def _():
                    start(i + _NBUF, b)

        for b in range(_NBUF):
            out_desc(0, b).wait()

    return body


def _make_embed_kernel():
    mesh = plsc.VectorSubcoreMesh(
        core_axis_name="c", subcore_axis_name="s",
        num_cores=_NC, num_subcores=_NS,
    )

    @functools.partial(
        pl.kernel,
        out_type=jax.ShapeDtypeStruct((_S, _D, _B), jnp.float32),
        mesh=mesh,
        scratch_types=[
            pltpu.VMEM((_S, _CHUNK), jnp.int32),
            pltpu.VMEM((_NBUF, _CHUNK), jnp.int32),
            pltpu.VMEM((_NBUF, _CHUNK), jnp.int32),
            pltpu.VMEM((_S * _D,), jnp.float32),
            pltpu.VMEM((_NBUF, _CHUNK, _CHUNK), jnp.float32),
            pltpu.VMEM((_NBUF, _D, _CHUNK), jnp.float32),
            pltpu.SemaphoreType.DMA,
            pltpu.SemaphoreType.DMA,
            pltpu.SemaphoreType.DMA,
            pltpu.SemaphoreType.DMA,
        ],
        compiler_params=pltpu.CompilerParams(
            use_tc_tiling_on_sc=True, needs_layout_passes=False),
    )
    def body(yt_hbm, pos_hbm, tab_hbm, out_hbm, idx_v, ix2_v, par_v, pos_v,
             buf_v, o_v, semi0, semi1, semo0, semo1):
        semi = (semi0, semi1)
        semo = (semo0, semo1)
        wid = lax.axis_index("s") * _NC + lax.axis_index("c")
        b0 = wid * _BPW
        pltpu.sync_copy(yt_hbm.at[:, pl.ds(b0, _BPW)], idx_v)
        pltpu.sync_copy(pos_hbm, pos_v)

        def prep_idx(s, b):
            for g in range(_CHUNK // _L):
                sl = pl.ds(g * _L, _L)
                iv = idx_v[s, sl]
                ix2_v[b, sl] = iv >> 1
                par_v[b, sl] = (iv & 1) << 6

        def start(s, b):
            pltpu.async_copy(
                tab_hbm.at[ix2_v.at[b]], buf_v.at[b], semi[b])

        def wait_in(b):
            pltpu.make_async_copy(
                tab_hbm.at[ix2_v.at[b]], buf_v.at[b], semi[b]).wait()

        def out_desc(s, b):
            return pltpu.make_async_copy(
                o_v.at[b], out_hbm.at[s, :, pl.ds(b0, _BPW)], semo[b])

        for b in range(_NBUF):
            prep_idx(b, b)
            start(b, b)

        iota = lax.iota(jnp.int32, _L)
        rot = [(iota + k) & (_L - 1) for k in range(_L)]

        @pl.loop(0, _S, step=_NBUF)
        def _chunks(c):
            for b in range(_NBUF):
                s = c + b
                wait_in(b)

                @pl.when(s >= _NBUF)
                def _():
                    out_desc(s - _NBUF, b).wait()

                for db in range(_D // _L):
                    pbase = jnp.full((_L,), s * _D + db * _L, jnp.int32)
                    pvs = [plsc.load_gather(pos_v, [pbase + rot[k]])
                           for k in range(_L)]

                    @pl.loop(0, _CHUNK // _L)
                    def _jb(jb):
                        jvec = jnp.full((_L,), _L, jnp.int32) * jb + iota
                        parv = par_v[b, pl.ds(jb * _L, _L)]
                        for k in range(_L):
                            drow = rot[k] + (db * _L)
                            v = plsc.load_gather(
                                buf_v.at[b], [jvec, parv + drow])
                            plsc.store_scatter(
                                o_v.at[b], [drow, jvec], v * 8.0 + pvs[k])

                out_desc(s, b).start()

                nxt = s + _NBUF

                @pl.when(nxt < _S)
                def _():
                    prep_idx(nxt, b)
                    start(nxt, b)

        for b in range(_NBUF):
            out_desc(0, b).wait()

    return body


_FORMAT_KERNEL = _make_format_kernel()
_EMBED_KERNEL = _make_embed_kernel()
_POS8 = _pos_enc8()


def kernel(y, lens, emb):
    table = _FORMAT_KERNEL(emb.T)
    out3 = _EMBED_KERNEL(y.T, jnp.asarray(_POS8), table)
    return jnp.transpose(out3, (2, 0, 1)), lens


# speedup vs baseline: 3.2006x; 1.5387x over previous
import functools

import numpy as np
import jax
import jax.numpy as jnp
from jax import lax
from jax.experimental import pallas as pl
from jax.experimental.pallas import tpu as pltpu
from jax.experimental.pallas import tpu_sc as plsc

_B, _S, _D = 4096, 200, 64
_N = _B * _S
_V = 1000000
_NC, _NS, _L = 2, 16, 16
_NW = _NC * _NS
_CHUNK = 128
_NT = (_V + _CHUNK - 1) // _CHUNK
_VP = _NT * _CHUNK // 2
_NBUF = 2
_BPW = _B // _NW


def _pos_enc8() -> np.ndarray:
    d = np.arange(_D)[np.newaxis, :]
    d = 1.0 / np.power(10000, 2 * (d // 2) / np.float32(_D))
    t = np.arange(_S)[:, np.newaxis] * d
    t = np.concatenate([np.sin(t[:, 0::2]), np.cos(t[:, 1::2])], axis=-1)
    return (t * 8.0).astype(np.float32).reshape(-1)


def _make_format_kernel():
    mesh = plsc.VectorSubcoreMesh(
        core_axis_name="c", subcore_axis_name="s",
        num_cores=_NC, num_subcores=_NS,
    )

    @functools.partial(
        pl.kernel,
        out_type=jax.ShapeDtypeStruct((_VP, _CHUNK), jnp.float32),
        mesh=mesh,
        scratch_types=[
            pltpu.VMEM((_NBUF, _D, _CHUNK), jnp.float32),
            pltpu.VMEM((_NBUF, _D, _CHUNK), jnp.float32),
            pltpu.SemaphoreType.DMA,
            pltpu.SemaphoreType.DMA,
            pltpu.SemaphoreType.DMA,
            pltpu.SemaphoreType.DMA,
        ],
        compiler_params=pltpu.CompilerParams(
            use_tc_tiling_on_sc=True, disable_bounds_checks=True,
            needs_layout_passes=False),
    )
    def body(embt_hbm, tab_hbm, e_v, t_v, semi0, semi1, semo0, semo1):
        semi = (semi0, semi1)
        semo = (semo0, semo1)
        wid = lax.axis_index("s") * _NC + lax.axis_index("c")
        nt = (_NT + _NW - 1 - wid) // _NW

        def start(i, b):
            t = wid + i * _NW
            pltpu.async_copy(
                embt_hbm.at[:, pl.ds(t * _CHUNK, _CHUNK)], e_v.at[b],
                semi[b])

        def wait_in(i, b):
            t = wid + i * _NW
            pltpu.make_async_copy(
                embt_hbm.at[:, pl.ds(t * _CHUNK, _CHUNK)], e_v.at[b],
                semi[b]).wait()

        def out_desc(i, b):
            t = wid + i * _NW
            return pltpu.make_async_copy(
                t_v.at[b], tab_hbm.at[pl.ds(t * _D, _D)], semo[b])

        for b in range(_NBUF):
            start(b, b)

        iota = lax.iota(jnp.int32, _L)
        rot = [(iota + k) & (_L - 1) for k in range(_L)]
        cvec = 2 * (iota & 7) + (iota >> 3)
        r8 = iota & 7
        h64 = (iota >> 3) * _D

        @pl.loop(0, 246, step=_NBUF)
        def _tiles(c):
            for b in range(_NBUF):
                i = c + b

                @pl.when(i < nt)
                def _():
                    wait_in(i, b)

                    @pl.when(i >= _NBUF)
                    def _():
                        out_desc(i - _NBUF, b).wait()

                    @pl.loop(0, 8)
                    def _rb(rb):
                        srccol = jnp.full((_L,), 2 * 8, jnp.int32) * rb + cvec
                        dstrow = jnp.full((_L,), 8, jnp.int32) * rb + r8
                        for db in range(_D // _L):
                            for k in range(_L):
                                srcrow = rot[k] + (db * _L)
                                v = plsc.load_gather(
                                    e_v.at[b], [srcrow, srccol])
                                plsc.store_scatter(
                                    t_v.at[b], [dstrow, h64 + srcrow], v)

                    out_desc(i, b).start()

                @pl.when(i + _NBUF < nt)
                def _():
                    start(i + _NBUF, b)

        for b in range(_NBUF):
            out_desc(0, b).wait()

    return body


def _make_embed_kernel():
    mesh = plsc.VectorSubcoreMesh(
        core_axis_name="c", subcore_axis_name="s",
        num_cores=_NC, num_subcores=_NS,
    )

    @functools.partial(
        pl.kernel,
        out_type=jax.ShapeDtypeStruct((_S, _D, _B), jnp.float32),
        mesh=mesh,
        scratch_types=[
            pltpu.VMEM((_S, _CHUNK), jnp.int32),
            pltpu.VMEM((_NBUF, _CHUNK), jnp.int32),
            pltpu.VMEM((_NBUF, _CHUNK), jnp.int32),
            pltpu.VMEM((_S * _D,), jnp.float32),
            pltpu.VMEM((_NBUF, _CHUNK, _CHUNK), jnp.float32),
            pltpu.VMEM((_NBUF, _D, _CHUNK), jnp.float32),
            pltpu.SemaphoreType.DMA,
            pltpu.SemaphoreType.DMA,
            pltpu.SemaphoreType.DMA,
            pltpu.SemaphoreType.DMA,
        ],
        compiler_params=pltpu.CompilerParams(
            use_tc_tiling_on_sc=True, needs_layout_passes=False),
    )
    def body(yt_hbm, pos_hbm, tab_hbm, out_hbm, idx_v, ix2_v, par_v, pos_v,
             buf_v, o_v, semi0, semi1, semo0, semo1):
        semi = (semi0, semi1)
        semo = (semo0, semo1)
        wid = lax.axis_index("s") * _NC + lax.axis_index("c")
        b0 = wid * _BPW
        pltpu.sync_copy(yt_hbm.at[:, pl.ds(b0, _BPW)], idx_v)
        pltpu.sync_copy(pos_hbm, pos_v)

        def prep_idx(s, b):
            for g in range(_CHUNK // _L):
                sl = pl.ds(g * _L, _L)
                iv = idx_v[s, sl]
                ix2_v[b, sl] = iv >> 1
                par_v[b, sl] = (iv & 1) << 6

        def start(s, b):
            pltpu.async_copy(
                tab_hbm.at[ix2_v.at[b]], buf_v.at[b], semi[b])

        def wait_in(b):
            pltpu.make_async_copy(
                tab_hbm.at[ix2_v.at[b]], buf_v.at[b], semi[b]).wait()

        def out_desc(s, b):
            return pltpu.make_async_copy(
                o_v.at[b], out_hbm.at[s, :, pl.ds(b0, _BPW)], semo[b])

        for b in range(_NBUF):
            prep_idx(b, b)
            start(b, b)

        iota = lax.iota(jnp.int32, _L)
        rot = [(iota + k) & (_L - 1) for k in range(_L)]

        @pl.loop(0, _S, step=_NBUF)
        def _chunks(c):
            for b in range(_NBUF):
                s = c + b
                wait_in(b)

                @pl.when(s >= _NBUF)
                def _():
                    out_desc(s - _NBUF, b).wait()

                for db in range(_D // _L):
                    pbase = jnp.full((_L,), s * _D + db * _L, jnp.int32)
                    pvs = [plsc.load_gather(pos_v, [pbase + rot[k]])
                           for k in range(_L)]
                    drows = [rot[k] + (db * _L) for k in range(_L)]

                    @pl.loop(0, _CHUNK // _L)
                    def _jb(jb):
                        jvec = jnp.full((_L,), _L, jnp.int32) * jb + iota
                        parv = par_v[b, pl.ds(jb * _L, _L)]
                        vals = [plsc.load_gather(
                                    buf_v.at[b], [jvec, parv + drows[k]])
                                for k in range(_L)]
                        for k in range(_L):
                            plsc.store_scatter(
                                o_v.at[b], [drows[k], jvec],
                                vals[k] * 8.0 + pvs[k])

                out_desc(s, b).start()

                nxt = s + _NBUF

                @pl.when(nxt < _S)
                def _():
                    prep_idx(nxt, b)
                    start(nxt, b)

        for b in range(_NBUF):
            out_desc(0, b).wait()

    return body


_FORMAT_KERNEL = _make_format_kernel()
_EMBED_KERNEL = _make_embed_kernel()
_POS8 = _pos_enc8()


def kernel(y, lens, emb):
    table = emb.reshape(_V // 2, 2 * _D)
    out3 = _EMBED_KERNEL(y.T, jnp.asarray(_POS8), table)
    return jnp.transpose(out3, (2, 0, 1)), lens


# speedup vs baseline: 4.7447x; 1.4824x over previous
import functools

import numpy as np
import jax
import jax.numpy as jnp
from jax import lax
from jax.experimental import pallas as pl
from jax.experimental.pallas import tpu as pltpu
from jax.experimental.pallas import tpu_sc as plsc

_B, _S, _D = 4096, 200, 64
_N = _B * _S
_V = 1000000
_NC, _NS, _L = 2, 16, 16
_NW = _NC * _NS
_CHUNK = 128
_NT = (_V + _CHUNK - 1) // _CHUNK
_VP = _NT * _CHUNK // 2
_NBUF = 2
_BPW = _B // _NW


def _pos_enc8() -> np.ndarray:
    d = np.arange(_D)[np.newaxis, :]
    d = 1.0 / np.power(10000, 2 * (d // 2) / np.float32(_D))
    t = np.arange(_S)[:, np.newaxis] * d
    t = np.concatenate([np.sin(t[:, 0::2]), np.cos(t[:, 1::2])], axis=-1)
    return (t * 8.0).astype(np.float32).reshape(-1)


def _make_format_kernel():
    mesh = plsc.VectorSubcoreMesh(
        core_axis_name="c", subcore_axis_name="s",
        num_cores=_NC, num_subcores=_NS,
    )

    @functools.partial(
        pl.kernel,
        out_type=jax.ShapeDtypeStruct((_VP, _CHUNK), jnp.float32),
        mesh=mesh,
        scratch_types=[
            pltpu.VMEM((_NBUF, _D, _CHUNK), jnp.float32),
            pltpu.VMEM((_NBUF, _D, _CHUNK), jnp.float32),
            pltpu.SemaphoreType.DMA,
            pltpu.SemaphoreType.DMA,
            pltpu.SemaphoreType.DMA,
            pltpu.SemaphoreType.DMA,
        ],
        compiler_params=pltpu.CompilerParams(
            use_tc_tiling_on_sc=True, disable_bounds_checks=True,
            needs_layout_passes=False),
    )
    def body(embt_hbm, tab_hbm, e_v, t_v, semi0, semi1, semo0, semo1):
        semi = (semi0, semi1)
        semo = (semo0, semo1)
        wid = lax.axis_index("s") * _NC + lax.axis_index("c")
        nt = (_NT + _NW - 1 - wid) // _NW

        def start(i, b):
            t = wid + i * _NW
            pltpu.async_copy(
                embt_hbm.at[:, pl.ds(t * _CHUNK, _CHUNK)], e_v.at[b],
                semi[b])

        def wait_in(i, b):
            t = wid + i * _NW
            pltpu.make_async_copy(
                embt_hbm.at[:, pl.ds(t * _CHUNK, _CHUNK)], e_v.at[b],
                semi[b]).wait()

        def out_desc(i, b):
            t = wid + i * _NW
            return pltpu.make_async_copy(
                t_v.at[b], tab_hbm.at[pl.ds(t * _D, _D)], semo[b])

        for b in range(_NBUF):
            start(b, b)

        iota = lax.iota(jnp.int32, _L)
        rot = [(iota + k) & (_L - 1) for k in range(_L)]
        cvec = 2 * (iota & 7) + (iota >> 3)
        r8 = iota & 7
        h64 = (iota >> 3) * _D

        @pl.loop(0, 246, step=_NBUF)
        def _tiles(c):
            for b in range(_NBUF):
                i = c + b

                @pl.when(i < nt)
                def _():
                    wait_in(i, b)

                    @pl.when(i >= _NBUF)
                    def _():
                        out_desc(i - _NBUF, b).wait()

                    @pl.loop(0, 8)
                    def _rb(rb):
                        srccol = jnp.full((_L,), 2 * 8, jnp.int32) * rb + cvec
                        dstrow = jnp.full((_L,), 8, jnp.int32) * rb + r8
                        for db in range(_D // _L):
                            srows = [rot[k] + (db * _L) for k in range(_L)]
                            vals = [plsc.load_gather(
                                        e_v.at[b], [srows[k], srccol])
                                    for k in range(_L)]
                            for k in range(_L):
                                plsc.store_scatter(
                                    t_v.at[b], [dstrow, h64 + srows[k]],
                                    vals[k])

                    out_desc(i, b).start()

                @pl.when(i + _NBUF < nt)
                def _():
                    start(i + _NBUF, b)

        for b in range(_NBUF):
            out_desc(0, b).wait()

    return body


def _make_embed_kernel():
    mesh = plsc.VectorSubcoreMesh(
        core_axis_name="c", subcore_axis_name="s",
        num_cores=_NC, num_subcores=_NS,
    )

    @functools.partial(
        pl.kernel,
        out_type=jax.ShapeDtypeStruct((_S, _D, _B), jnp.float32),
        mesh=mesh,
        scratch_types=[
            pltpu.VMEM((_S, _CHUNK), jnp.int32),
            pltpu.VMEM((_NBUF, _CHUNK), jnp.int32),
            pltpu.VMEM((_NBUF, _CHUNK), jnp.int32),
            pltpu.VMEM((_S * _D,), jnp.float32),
            pltpu.VMEM((_NBUF, _CHUNK, _CHUNK), jnp.float32),
            pltpu.VMEM((_NBUF, _D, _CHUNK), jnp.float32),
            pltpu.SemaphoreType.DMA,
            pltpu.SemaphoreType.DMA,
            pltpu.SemaphoreType.DMA,
            pltpu.SemaphoreType.DMA,
        ],
        compiler_params=pltpu.CompilerParams(
            use_tc_tiling_on_sc=True, needs_layout_passes=False),
    )
    def body(yt_hbm, pos_hbm, tab_hbm, out_hbm, idx_v, ix2_v, par_v, pos_v,
             buf_v, o_v, semi0, semi1, semo0, semo1):
        semi = (semi0, semi1)
        semo = (semo0, semo1)
        wid = lax.axis_index("s") * _NC + lax.axis_index("c")
        b0 = wid * _BPW
        pltpu.sync_copy(yt_hbm.at[:, pl.ds(b0, _BPW)], idx_v)
        pltpu.sync_copy(pos_hbm, pos_v)

        def prep_idx(s, b):
            for g in range(_CHUNK // _L):
                sl = pl.ds(g * _L, _L)
                iv = idx_v[s, sl]
                ix2_v[b, sl] = iv >> 1
                par_v[b, sl] = (iv & 1) << 6

        def start(s, b):
            pltpu.async_copy(
                tab_hbm.at[ix2_v.at[b]], buf_v.at[b], semi[b])

        def wait_in(b):
            pltpu.make_async_copy(
                tab_hbm.at[ix2_v.at[b]], buf_v.at[b], semi[b]).wait()

        def out_desc(s, b):
            return pltpu.make_async_copy(
                o_v.at[b], out_hbm.at[s, :, pl.ds(b0, _BPW)], semo[b])

        for b in range(_NBUF):
            prep_idx(b, b)
            start(b, b)

        iota = lax.iota(jnp.int32, _L)
        rot = [(iota + k) & (_L - 1) for k in range(_L)]

        @pl.loop(0, _S, step=_NBUF)
        def _chunks(c):
            for b in range(_NBUF):
                s = c + b
                wait_in(b)

                @pl.when(s >= _NBUF)
                def _():
                    out_desc(s - _NBUF, b).wait()

                for db in range(_D // _L):
                    pbase = jnp.full((_L,), s * _D + db * _L, jnp.int32)
                    pvs = [plsc.load_gather(pos_v, [pbase + rot[k]])
                           for k in range(_L)]
                    drows = [rot[k] + (db * _L) for k in range(_L)]

                    @pl.loop(0, _CHUNK // _L)
                    def _jb(jb):
                        jvec = jnp.full((_L,), _L, jnp.int32) * jb + iota
                        parv = par_v[b, pl.ds(jb * _L, _L)]
                        vals = [plsc.load_gather(
                                    buf_v.at[b], [jvec, parv + drows[k]])
                                for k in range(_L)]
                        for k in range(_L):
                            plsc.store_scatter(
                                o_v.at[b], [drows[k], jvec],
                                vals[k] * 8.0 + pvs[k])

                out_desc(s, b).start()

                nxt = s + _NBUF

                @pl.when(nxt < _S)
                def _():
                    prep_idx(nxt, b)
                    start(nxt, b)

        for b in range(_NBUF):
            out_desc(0, b).wait()

    return body


_FORMAT_KERNEL = _make_format_kernel()
_EMBED_KERNEL = _make_embed_kernel()
_POS8 = _pos_enc8()


def kernel(y, lens, emb):
    table = _FORMAT_KERNEL(emb.T)
    out3 = _EMBED_KERNEL(y.T, jnp.asarray(_POS8), table)
    return jnp.transpose(out3, (2, 0, 1)), lens
